# baseline (speedup 1.0000x reference)
import functools

import jax
import jax.numpy as jnp
from jax import lax
from jax.experimental import pallas as pl
from jax.experimental.pallas import tpu as pltpu
from jax.experimental.pallas import tpu_sc as plsc

_RES = 128
_N, _H, _W = 8, 256, 256
_PIX = _H * _W
_GRID = _RES ** 3
_NW = 32
_SHARD = _GRID // _NW
_CHUNK = 8192
_NCHUNK = _PIX // _CHUNK
_LANES = 16


def _project_body(depth_ref, fl_ref, cd_ref, idx_ref, val_ref):
    depth = depth_ref[0]
    b = pl.program_id(0)
    fl = fl_ref[b, 0]
    cd = cd_ref[b, 0]
    res = float(_RES)
    u = lax.broadcasted_iota(jnp.int32, (_H, _W), 1).astype(jnp.float32) \
        - (_W / 2.0 - 0.5)
    v = lax.broadcasted_iota(jnp.int32, (_H, _W), 0).astype(jnp.float32) \
        - (_H / 2.0 - 0.5)
    norm = jnp.sqrt(u * u + v * v + fl * fl)
    x = depth * u / norm
    y = depth * v / norm
    z = cd - depth * fl / norm
    ix = jnp.floor((x + 0.5) * res)
    iy = jnp.floor((y + 0.5) * res)
    iz = jnp.floor((z + 0.5) * res)
    cx = (ix + 0.5) / res - 0.5
    cy = (iy + 0.5) / res - 0.5
    cz = (iz + 0.5) / res - 0.5
    dist = jnp.sqrt((x - cx) ** 2 + (y - cy) ** 2 + (z - cz) ** 2 + 1e-12)
    valid = ((ix >= 0) & (ix < res) & (iy >= 0) & (iy < res)
             & (iz >= 0) & (iz < res))
    flat = (ix.astype(jnp.int32) * _RES + iy.astype(jnp.int32)) * _RES \
        + iz.astype(jnp.int32)
    idx_ref[0] = jnp.where(valid, flat, -1)
    val_ref[0] = 1.0 - res * dist


_project = pl.pallas_call(
    _project_body,
    grid=(_N,),
    in_specs=[
        pl.BlockSpec((1, _H, _W), lambda b: (b, 0, 0)),
        pl.BlockSpec((_N, 1), lambda b: (0, 0), memory_space=pltpu.SMEM),
        pl.BlockSpec((_N, 1), lambda b: (0, 0), memory_space=pltpu.SMEM),
    ],
    out_specs=[
        pl.BlockSpec((1, _H, _W), lambda b: (b, 0, 0)),
        pl.BlockSpec((1, _H, _W), lambda b: (b, 0, 0)),
    ],
    out_shape=[
        jax.ShapeDtypeStruct((_N, _H, _W), jnp.int32),
        jax.ShapeDtypeStruct((_N, _H, _W), jnp.float32),
    ],
)


def _scatter_body(idx_hbm, val_hbm, out_hbm, shard, idx_buf, val_buf):
    wid = lax.axis_index("s") * 2 + lax.axis_index("c")
    lo = wid * _SHARD

    def batch_body(b, carry):
        def init_body(i, c):
            shard[pl.ds(i * _LANES, _LANES)] = jnp.zeros((_LANES,), jnp.float32)
            return c
        lax.fori_loop(0, _SHARD // _LANES, init_body, 0)

        def chunk_body(ch, c):
            pltpu.sync_copy(idx_hbm.at[b, pl.ds(ch * _CHUNK, _CHUNK)], idx_buf)
            pltpu.sync_copy(val_hbm.at[b, pl.ds(ch * _CHUNK, _CHUNK)], val_buf)

            def vec_body(j, cc):
                idx = idx_buf[pl.ds(j * _LANES, _LANES)]
                val = val_buf[pl.ds(j * _LANES, _LANES)]
                local = idx - lo
                mask = (idx >= lo) & (local < _SHARD)
                safe = jnp.where(mask, local, 0)
                cur = plsc.load_gather(shard, [safe], mask=mask)
                need = mask & (val > cur)

                def w_cond(n):
                    return jnp.any(n)

                def w_body(n):
                    plsc.store_scatter(shard, [safe], val, mask=n)
                    cur2 = plsc.load_gather(shard, [safe], mask=n)
                    return n & (val > cur2)

                lax.while_loop(w_cond, w_body, need)
                return cc
            lax.fori_loop(0, _CHUNK // _LANES, vec_body, 0)
            return c
        lax.fori_loop(0, _NCHUNK, chunk_body, 0)
        pltpu.sync_copy(shard, out_hbm.at[b, pl.ds(lo, _SHARD)])
        return carry
    lax.fori_loop(0, _N, batch_body, 0)


@functools.lru_cache(maxsize=1)
def _build_scatter_max():
    mesh = plsc.VectorSubcoreMesh(
        core_axis_name="c", subcore_axis_name="s",
        num_cores=2, num_subcores=16)
    return pl.kernel(
        _scatter_body,
        out_type=jax.ShapeDtypeStruct((_N, _GRID), jnp.float32),
        mesh=mesh,
        compiler_params=pltpu.CompilerParams(needs_layout_passes=False),
        scratch_types=[
            pltpu.VMEM((_SHARD,), jnp.float32),
            pltpu.VMEM((_CHUNK,), jnp.int32),
            pltpu.VMEM((_CHUNK,), jnp.float32),
        ],
    )


@jax.jit
def kernel(depth_t, fl, cam_dist):
    depth2d = depth_t.reshape(_N, _H, _W)
    idx, val = _project(depth2d, fl, cam_dist)
    grid = _build_scatter_max()(idx.reshape(_N, _PIX), val.reshape(_N, _PIX))
    return grid.reshape(_N, 1, _RES, _RES, _RES)
